# Initial kernel scaffold; baseline (speedup 1.0000x reference)
import jax
import jax.numpy as jnp
from jax.experimental import pallas as pl


def kernel(X, W0, b0, g0, beta0, W1, b1, g1, beta1, W2, b2, g2, beta2, Wsc, gsc, betasc):
    raise NotImplementedError("write your pallas kernel here")


# speedup vs baseline: 3.1032x; 3.1032x over previous
import functools

import jax
import jax.numpy as jnp
from jax import lax
from jax.experimental import pallas as pl
from jax.experimental.pallas import tpu as pltpu
from jax.experimental.pallas import tpu_sc as plsc

_K = 16
_EPS = 1e-3
_NC, _NS = 2, 16
_NW = _NC * _NS


def _knn_kernel(x_ref, xt_ref, nn_ref, d_scr):
    RB, N = d_scr.shape
    d = jnp.zeros((RB, N), jnp.float32)
    for f in range(3):
        pi = xt_ref[0, :, f:f + 1]
        pj = x_ref[0, f:f + 1, :]
        diff = pi - pj
        d = d + diff * diff
    d_scr[:, :] = d
    lane = lax.broadcasted_iota(jnp.int32, (RB, N), 1)
    big = jnp.float32(3.0e38)
    for j in range(_K + 1):
        dd = d_scr[:, :]
        m = jnp.min(dd, axis=1, keepdims=True)
        idx = jnp.min(jnp.where(dd == m, lane, N), axis=1, keepdims=True)
        d_scr[:, :] = jnp.where(lane == idx, big, dd)
        if j > 0:
            nn_ref[0, :, j - 1:j] = idx


def _knn(X, Xt3, interpret=False):
    B, F, N = X.shape
    RB = 256
    grid = (B, N // RB)
    return pl.pallas_call(
        _knn_kernel,
        grid=grid,
        in_specs=[
            pl.BlockSpec((1, 8, N), lambda b, r: (b, 0, 0)),
            pl.BlockSpec((1, RB, F), lambda b, r: (b, r, 0)),
        ],
        out_specs=pl.BlockSpec((1, RB, 128), lambda b, r: (b, r, 0)),
        out_shape=jax.ShapeDtypeStruct((B, N, 128), jnp.int32),
        scratch_shapes=[pltpu.VMEM((RB, N), jnp.float32)],
        interpret=interpret,
    )(X, Xt3)


def _sc_gather(table, idx, E, D):
    RW = E // _NW
    CH = 128
    G = 4
    n_chunks = RW // CH
    NIT = n_chunks // G
    mesh = plsc.VectorSubcoreMesh(core_axis_name="c", subcore_axis_name="s")

    @functools.partial(
        pl.kernel,
        mesh=mesh,
        out_type=jax.ShapeDtypeStruct((E, D), jnp.float32),
        scratch_types=[
            pltpu.VMEM((G, CH), jnp.int32),
            pltpu.VMEM((G, CH, D), jnp.float32),
            pltpu.SemaphoreType.DMA((G,)),
        ],
    )
    def gk(table_hbm, idx_hbm, out_hbm, idx_v, rows_v, sem):
        wid = lax.axis_index("s") * _NC + lax.axis_index("c")
        base = wid * RW

        def issue(slot, chunk):
            pltpu.sync_copy(idx_hbm.at[pl.ds(base + chunk * CH, CH)],
                            idx_v.at[slot])
            pltpu.async_copy(table_hbm.at[idx_v.at[slot]], rows_v.at[slot],
                             sem.at[slot])

        for g in range(G):
            issue(g, g)

        def outer(t, carry):
            for g in range(G):
                c = t * G + g
                pltpu.make_async_copy(table_hbm.at[idx_v.at[g]],
                                      rows_v.at[g], sem.at[g]).wait()
                pltpu.sync_copy(rows_v.at[g],
                                out_hbm.at[pl.ds(base + c * CH, CH)])

                @pl.when(t < NIT - 1)
                def _():
                    issue(g, c + G)
            return carry

        lax.fori_loop(0, NIT, outer, 0)

    return gk(table, idx)


def _l0_kernel(xg_ref, xt_ref, w0at_ref, w0bat_ref, b0_ref, z0_ref, st_ref,
               c_scr):
    b = pl.program_id(0)
    k = pl.program_id(1)

    @pl.when(k == 0)
    def _():
        c_scr[:, :] = (jnp.dot(xt_ref[0], w0bat_ref[:, :],
                               preferred_element_type=jnp.float32)
                       + b0_ref[:, :])

    ag = jnp.dot(xg_ref[0, 0], w0at_ref[:, :],
                 preferred_element_type=jnp.float32)
    z = jnp.maximum(ag + c_scr[:, :], 0.0)
    z0_ref[0, 0] = z

    @pl.when(jnp.logical_and(b == 0, k == 0))
    def _():
        st_ref[:, :] = jnp.zeros_like(st_ref)

    st_ref[0:1, :] += jnp.sum(z, axis=0, keepdims=True)
    st_ref[1:2, :] += jnp.sum(z * z, axis=0, keepdims=True)


def _l0(Xg4, Xt3, W0at, W0bat, b0r, interpret=False):
    B, K, N, FP = Xg4.shape
    F = Xt3.shape[2]
    C = W0at.shape[1]
    return pl.pallas_call(
        _l0_kernel,
        grid=(B, K),
        in_specs=[
            pl.BlockSpec((1, 1, N, FP), lambda b, k: (b, k, 0, 0)),
            pl.BlockSpec((1, N, F), lambda b, k: (b, 0, 0)),
            pl.BlockSpec((FP, C), lambda b, k: (0, 0)),
            pl.BlockSpec((F, C), lambda b, k: (0, 0)),
            pl.BlockSpec((1, C), lambda b, k: (0, 0)),
        ],
        out_specs=[
            pl.BlockSpec((1, 1, N, C), lambda b, k: (b, k, 0, 0)),
            pl.BlockSpec((8, C), lambda b, k: (0, 0)),
        ],
        out_shape=[
            jax.ShapeDtypeStruct((B, K, N, C), jnp.float32),
            jax.ShapeDtypeStruct((8, C), jnp.float32),
        ],
        scratch_shapes=[pltpu.VMEM((N, C), jnp.float32)],
        interpret=interpret,
    )(Xg4, Xt3, W0at, W0bat, b0r)


def _mid_kernel(cnt, z_ref, st_in_ref, wt_ref, br_ref, g_ref, be_ref,
                zo_ref, st_ref):
    b = pl.program_id(0)
    k = pl.program_id(1)
    s = st_in_ref[0:1, :]
    ss = st_in_ref[1:2, :]
    m = s * (1.0 / cnt)
    var = ss * (1.0 / cnt) - m * m
    scale = g_ref[:, :] * lax.rsqrt(var + _EPS)
    shift = be_ref[:, :] - m * scale
    zin = z_ref[0, 0] * scale + shift
    z = jnp.maximum(jnp.dot(zin, wt_ref[:, :],
                            preferred_element_type=jnp.float32)
                    + br_ref[:, :], 0.0)
    zo_ref[0, 0] = z

    @pl.when(jnp.logical_and(b == 0, k == 0))
    def _():
        st_ref[:, :] = jnp.zeros_like(st_ref)

    st_ref[0:1, :] += jnp.sum(z, axis=0, keepdims=True)
    st_ref[1:2, :] += jnp.sum(z * z, axis=0, keepdims=True)


def _l1(Z0, st0, W1t, b1r, g0r, be0r, interpret=False):
    B, K, N, C = Z0.shape
    cnt = float(B * K * N)
    return pl.pallas_call(
        functools.partial(_mid_kernel, cnt),
        grid=(B, K),
        in_specs=[
            pl.BlockSpec((1, 1, N, C), lambda b, k: (b, k, 0, 0)),
            pl.BlockSpec((8, C), lambda b, k: (0, 0)),
            pl.BlockSpec((C, C), lambda b, k: (0, 0)),
            pl.BlockSpec((1, C), lambda b, k: (0, 0)),
            pl.BlockSpec((1, C), lambda b, k: (0, 0)),
            pl.BlockSpec((1, C), lambda b, k: (0, 0)),
        ],
        out_specs=[
            pl.BlockSpec((1, 1, N, C), lambda b, k: (b, k, 0, 0)),
            pl.BlockSpec((8, C), lambda b, k: (0, 0)),
        ],
        out_shape=[
            jax.ShapeDtypeStruct((B, K, N, C), jnp.float32),
            jax.ShapeDtypeStruct((8, C), jnp.float32),
        ],
        interpret=interpret,
    )(Z0, st0, W1t, b1r, g0r, be0r)


def _l2_kernel(cnt, z_ref, st_in_ref, wt_ref, br_ref, g_ref, be_ref,
               x_ref, wsc_ref, s2_ref, st_ref, sc_ref, stsc_ref):
    b = pl.program_id(0)
    k = pl.program_id(1)
    s = st_in_ref[0:1, :]
    ss = st_in_ref[1:2, :]
    m = s * (1.0 / cnt)
    var = ss * (1.0 / cnt) - m * m
    scale = g_ref[:, :] * lax.rsqrt(var + _EPS)
    shift = be_ref[:, :] - m * scale
    zin = z_ref[0, 0] * scale + shift
    z = jnp.maximum(jnp.dot(zin, wt_ref[:, :],
                            preferred_element_type=jnp.float32)
                    + br_ref[:, :], 0.0)

    @pl.when(k == 0)
    def _():
        s2_ref[0] = z
        scv = jnp.dot(wsc_ref[:, :], x_ref[0],
                      preferred_element_type=jnp.float32)
        sc_ref[0] = scv

        @pl.when(b == 0)
        def _():
            stsc_ref[:, :] = jnp.zeros_like(stsc_ref)

        stsc_ref[:, 0:1] += jnp.sum(scv, axis=1, keepdims=True)
        stsc_ref[:, 1:2] += jnp.sum(scv * scv, axis=1, keepdims=True)

    @pl.when(k > 0)
    def _():
        s2_ref[0] += z

    @pl.when(jnp.logical_and(b == 0, k == 0))
    def _():
        st_ref[:, :] = jnp.zeros_like(st_ref)

    st_ref[0:1, :] += jnp.sum(z, axis=0, keepdims=True)
    st_ref[1:2, :] += jnp.sum(z * z, axis=0, keepdims=True)


def _l2(Z1, st1, W2t, b2r, g1r, be1r, X, Wsc, interpret=False):
    B, K, N, C = Z1.shape
    F = X.shape[1]
    cnt = float(B * K * N)
    return pl.pallas_call(
        functools.partial(_l2_kernel, cnt),
        grid=(B, K),
        in_specs=[
            pl.BlockSpec((1, 1, N, C), lambda b, k: (b, k, 0, 0)),
            pl.BlockSpec((8, C), lambda b, k: (0, 0)),
            pl.BlockSpec((C, C), lambda b, k: (0, 0)),
            pl.BlockSpec((1, C), lambda b, k: (0, 0)),
            pl.BlockSpec((1, C), lambda b, k: (0, 0)),
            pl.BlockSpec((1, C), lambda b, k: (0, 0)),
            pl.BlockSpec((1, F, N), lambda b, k: (b, 0, 0)),
            pl.BlockSpec((C, F), lambda b, k: (0, 0)),
        ],
        out_specs=[
            pl.BlockSpec((1, N, C), lambda b, k: (b, 0, 0)),
            pl.BlockSpec((8, C), lambda b, k: (0, 0)),
            pl.BlockSpec((1, C, N), lambda b, k: (b, 0, 0)),
            pl.BlockSpec((C, 8), lambda b, k: (0, 0)),
        ],
        out_shape=[
            jax.ShapeDtypeStruct((B, N, C), jnp.float32),
            jax.ShapeDtypeStruct((8, C), jnp.float32),
            jax.ShapeDtypeStruct((B, C, N), jnp.float32),
            jax.ShapeDtypeStruct((C, 8), jnp.float32),
        ],
        interpret=interpret,
    )(Z1, st1, W2t, b2r, g1r, be1r, X, Wsc)


def _fin_kernel(cnt2, cntsc, s2_ref, sc_ref, st2_ref, stsc_ref, g2_ref,
                be2_ref, gsc_ref, besc_ref, out_ref):
    s = st2_ref[0:1, :]
    ss = st2_ref[1:2, :]
    m = s * (1.0 / cnt2)
    var = ss * (1.0 / cnt2) - m * m
    scale = g2_ref[:, :] * lax.rsqrt(var + _EPS)
    shift = be2_ref[:, :] - m * scale
    hp = s2_ref[0] * (1.0 / _K)
    hpn = hp * scale + shift
    hpt = jnp.transpose(hpn, (1, 0))

    s_c = stsc_ref[:, 0:1]
    ss_c = stsc_ref[:, 1:2]
    mc = s_c * (1.0 / cntsc)
    varc = ss_c * (1.0 / cntsc) - mc * mc
    scalec = gsc_ref[:, :] * lax.rsqrt(varc + _EPS)
    shiftc = besc_ref[:, :] - mc * scalec
    scn = sc_ref[0] * scalec + shiftc
    out_ref[0] = jnp.maximum(hpt + scn, 0.0)


def _fin(S2, SC, st2, stsc, g2r, be2r, gscc, bescc, interpret=False):
    B, N, C = S2.shape
    cnt2 = float(B * _K * N)
    cntsc = float(B * N)
    return pl.pallas_call(
        functools.partial(_fin_kernel, cnt2, cntsc),
        grid=(B,),
        in_specs=[
            pl.BlockSpec((1, N, C), lambda b: (b, 0, 0)),
            pl.BlockSpec((1, C, N), lambda b: (b, 0, 0)),
            pl.BlockSpec((8, C), lambda b: (0, 0)),
            pl.BlockSpec((C, 8), lambda b: (0, 0)),
            pl.BlockSpec((1, C), lambda b: (0, 0)),
            pl.BlockSpec((1, C), lambda b: (0, 0)),
            pl.BlockSpec((C, 1), lambda b: (0, 0)),
            pl.BlockSpec((C, 1), lambda b: (0, 0)),
        ],
        out_specs=pl.BlockSpec((1, C, N), lambda b: (b, 0, 0)),
        out_shape=jax.ShapeDtypeStruct((B, C, N), jnp.float32),
        interpret=interpret,
    )(S2, SC, st2, stsc, g2r, be2r, gscc, bescc)


def kernel(X, W0, b0, g0, beta0, W1, b1, g1, beta1, W2, b2, g2, beta2,
           Wsc, gsc, betasc):
    B, F, N = X.shape
    C = W0.shape[0]
    E = B * _K * N

    Xt = jnp.transpose(X, (0, 2, 1))
    nn = _knn(X, Xt)[:, :, :_K]
    idxT = jnp.transpose(nn, (0, 2, 1))
    flat_idx = (idxT
                + (jnp.arange(B, dtype=jnp.int32) * N)[:, None, None]
                ).reshape(E)
    FP = 128
    Xt_pad = jnp.concatenate(
        [Xt.reshape(B * N, F),
         jnp.zeros((B * N, FP - F), jnp.float32)], axis=1)
    Xg = _sc_gather(Xt_pad, flat_idx, E, FP)

    W0a = W0[:, :F]
    W0b = W0[:, F:]
    W0at_p = jnp.concatenate(
        [jnp.transpose(W0a), jnp.zeros((FP - F, C), jnp.float32)], axis=0)
    z0, st0 = _l0(Xg.reshape(B, _K, N, FP), Xt,
                  W0at_p, jnp.transpose(W0b - W0a),
                  b0.reshape(1, C))
    z1, st1 = _l1(z0, st0, jnp.transpose(W1), b1.reshape(1, C),
                  g0.reshape(1, C), beta0.reshape(1, C))
    s2, st2, sc, stsc = _l2(z1, st1, jnp.transpose(W2), b2.reshape(1, C),
                            g1.reshape(1, C), beta1.reshape(1, C), X, Wsc)
    return _fin(s2, sc, st2, stsc, g2.reshape(1, C), beta2.reshape(1, C),
                gsc.reshape(C, 1), betasc.reshape(C, 1))
